# baseline (speedup 1.0000x reference)
import functools

import jax
import jax.numpy as jnp
from jax import lax
from jax.experimental import pallas as pl
from jax.experimental.pallas import tpu as pltpu
from jax.experimental.pallas import tpu_sc as plsc

B = 4096
F = 26
V = 100000
D = 16

NC = 2
NS = 16
NW = NC * NS
BPW = B // NW
NIDX = F * BPW

_mesh = plsc.VectorSubcoreMesh(core_axis_name="c", subcore_axis_name="s")


@functools.partial(
    pl.kernel,
    mesh=_mesh,
    out_type=jax.ShapeDtypeStruct((B,), jnp.float32),
    scratch_types=[
        pltpu.VMEM((NIDX,), jnp.int32),
        pltpu.VMEM((NIDX, D), jnp.float32),
        pltpu.VMEM((BPW,), jnp.float32),
        pltpu.SemaphoreType.DMA,
    ],
    compiler_params=pltpu.CompilerParams(
        needs_layout_passes=False,
        use_tc_tiling_on_sc=False,
    ),
)
def _fm_sc(table_hbm, idx_hbm, out_hbm, idx_v, rows_v, out_v, sem):
    wid = lax.axis_index("s") * NC + lax.axis_index("c")
    base = wid * BPW

    pltpu.sync_copy(idx_hbm.at[wid], idx_v)

    copies = [
        pltpu.async_copy(
            table_hbm.at[f].at[idx_v.at[pl.ds(f * BPW, BPW)]],
            rows_v.at[pl.ds(f * BPW, BPW)],
            sem,
        )
        for f in range(F)
    ]
    for cp in copies:
        cp.wait()

    lane = lax.iota(jnp.int32, 16)

    def _group(g, carry):
        b0 = g * 16
        res = jnp.zeros((16,), jnp.float32)
        for j in range(16):
            s = jnp.zeros((D,), jnp.float32)
            ss = jnp.zeros((D,), jnp.float32)
            for f in range(F):
                v = rows_v[f * BPW + b0 + j, :]
                s = s + v
                ss = ss + v * v
            r = jnp.sum(s * s - ss)
            res = jnp.where(lane == j, r, res)
        out_v[pl.ds(b0, 16)] = 0.5 * res
        return carry

    lax.fori_loop(0, BPW // 16, _group, 0)

    pltpu.sync_copy(out_v, out_hbm.at[pl.ds(base, BPW)])


def kernel(X, tables):
    Xp = (
        X.astype(jnp.int32)
        .reshape(NW, BPW, F)
        .transpose(0, 2, 1)
        .reshape(NW, NIDX)
    )
    out = _fm_sc(tables, Xp)
    return out.reshape(B, 1)
